# baseline (speedup 1.0000x reference)
import functools

import jax
import jax.numpy as jnp
from jax import lax
from jax.experimental import pallas as pl
from jax.experimental.pallas import tpu as pltpu
from jax.experimental.pallas import tpu_sc as plsc

B = 16384
VOCAB = 1000000
EMBED = 64
HID = 256
OUT = 128

NC = 2
NS = 16
NW = NC * NS
B_PER_W = B // NW
CHUNK = 128
N_CHUNKS = B_PER_W // CHUNK
L = 16

_sc_mesh = plsc.VectorSubcoreMesh(core_axis_name="c", subcore_axis_name="s")


@functools.partial(
    pl.kernel,
    mesh=_sc_mesh,
    out_type=jax.ShapeDtypeStruct((B, 2 * EMBED), jnp.float32),
    scratch_types=[
        pltpu.VMEM((N_CHUNKS, CHUNK), jnp.int32),
        pltpu.VMEM((N_CHUNKS, CHUNK), jnp.int32),
        pltpu.VMEM((B_PER_W, 2 * EMBED), jnp.float32),
        pltpu.SemaphoreType.DMA,
    ],
)
def _sc_gather(idx_hbm, table_hbm, out_hbm, idx_v, idx2_v, rows_v, sem):
    wid = lax.axis_index("s") * NC + lax.axis_index("c")
    base = wid * B_PER_W
    pltpu.sync_copy(idx_hbm.at[wid], idx_v)
    for j in range(N_CHUNKS):
        for k in range(CHUNK // L):
            idx2_v[j, pl.ds(k * L, L)] = idx_v[j, pl.ds(k * L, L)] >> 1
    copies = []
    for j in range(N_CHUNKS):
        copies.append(
            pltpu.async_copy(
                table_hbm.at[idx2_v.at[j]],
                rows_v.at[pl.ds(j * CHUNK, CHUNK)],
                sem,
            )
        )
    for c in copies:
        c.wait()
    pltpu.sync_copy(rows_v, out_hbm.at[pl.ds(base, B_PER_W)])


def _mlp_body(x2_ref, idx_ref, w1_ref, b1_ref, w2_ref, b2_ref, o_ref):
    x2 = x2_ref[...]
    odd = (idx_ref[...] & 1) == 1
    x = jnp.where(odd, x2[:, EMBED:], x2[:, :EMBED])
    h = jnp.dot(x, w1_ref[...], preferred_element_type=jnp.float32)
    h = jnp.maximum(h + b1_ref[...], 0.0)
    o = jnp.dot(h, w2_ref[...], preferred_element_type=jnp.float32)
    o_ref[...] = jnp.maximum(o + b2_ref[...], 0.0)


BM = 2048


def _mlp(x2, idx, w1, b1, w2, b2):
    grid = (B // BM,)
    return pl.pallas_call(
        _mlp_body,
        out_shape=jax.ShapeDtypeStruct((B, OUT), jnp.float32),
        grid=grid,
        in_specs=[
            pl.BlockSpec((BM, 2 * EMBED), lambda i: (i, 0)),
            pl.BlockSpec((BM, 1), lambda i: (i, 0)),
            pl.BlockSpec((EMBED, HID), lambda i: (0, 0)),
            pl.BlockSpec((1, HID), lambda i: (0, 0)),
            pl.BlockSpec((HID, OUT), lambda i: (0, 0)),
            pl.BlockSpec((1, OUT), lambda i: (0, 0)),
        ],
        out_specs=pl.BlockSpec((BM, OUT), lambda i: (i, 0)),
    )(x2, idx, w1, b1, w2, b2)


@jax.jit
def kernel(indices, table, W1, b1, W2, b2):
    idx = indices.reshape(NW, N_CHUNKS, CHUNK)
    table2 = table.reshape(VOCAB // 2, 2 * EMBED)
    gathered = _sc_gather(idx, table2)
    return _mlp(
        gathered,
        indices.reshape(B, 1),
        W1,
        b1.reshape(1, HID),
        W2,
        b2.reshape(1, OUT),
    )


# speedup vs baseline: 1.7109x; 1.7109x over previous
import functools

import jax
import jax.numpy as jnp
from jax import lax
from jax.experimental import pallas as pl
from jax.experimental.pallas import tpu as pltpu
from jax.experimental.pallas import tpu_sc as plsc

B = 16384
VOCAB = 1000000
EMBED = 64
HID = 256
OUT = 128

NC = 2
NS = 16
NW = NC * NS
B_PER_W = B // NW

_sc_mesh = plsc.VectorSubcoreMesh(core_axis_name="c", subcore_axis_name="s")


@functools.partial(
    pl.kernel,
    mesh=_sc_mesh,
    out_type=jax.ShapeDtypeStruct((B, EMBED), jnp.float32),
    scratch_types=[
        pltpu.VMEM((B_PER_W,), jnp.int32),
        pltpu.VMEM((B_PER_W, EMBED), jnp.float32),
        pltpu.SemaphoreType.DMA,
    ],
)
def _sc_gather(idx_hbm, table_hbm, out_hbm, idx_s, rows_v, sem):
    wid = lax.axis_index("s") * NC + lax.axis_index("c")
    base = wid * B_PER_W
    pltpu.sync_copy(idx_hbm.at[wid], idx_s)

    def body(g, carry):
        v = idx_s[pl.ds(g * 16, 16)]
        for k in range(16):
            pltpu.async_copy(
                table_hbm.at[pl.ds(v[k], 1)],
                rows_v.at[pl.ds(g * 16 + k, 1)],
                sem,
            )
        return carry

    lax.fori_loop(0, B_PER_W // 16, body, 0)
    pltpu.make_async_copy(
        table_hbm.at[pl.ds(0, B_PER_W)],
        rows_v,
        sem,
    ).wait()
    pltpu.sync_copy(rows_v, out_hbm.at[pl.ds(base, B_PER_W)])


def _mlp_body(x2_ref, w1_ref, b1_ref, w2_ref, b2_ref, o_ref):
    x = x2_ref[...]
    h = jnp.dot(x, w1_ref[...], preferred_element_type=jnp.float32)
    h = jnp.maximum(h + b1_ref[...], 0.0)
    o = jnp.dot(h, w2_ref[...], preferred_element_type=jnp.float32)
    o_ref[...] = jnp.maximum(o + b2_ref[...], 0.0)


BM = 2048


def _mlp(x2, w1, b1, w2, b2):
    grid = (B // BM,)
    return pl.pallas_call(
        _mlp_body,
        out_shape=jax.ShapeDtypeStruct((B, OUT), jnp.float32),
        grid=grid,
        in_specs=[
            pl.BlockSpec((BM, EMBED), lambda i: (i, 0)),
            pl.BlockSpec((EMBED, HID), lambda i: (0, 0)),
            pl.BlockSpec((1, HID), lambda i: (0, 0)),
            pl.BlockSpec((HID, OUT), lambda i: (0, 0)),
            pl.BlockSpec((1, OUT), lambda i: (0, 0)),
        ],
        out_specs=pl.BlockSpec((BM, OUT), lambda i: (i, 0)),
    )(x2, w1, b1, w2, b2)


@jax.jit
def kernel(indices, table, W1, b1, W2, b2):
    idx = indices.reshape(NW, B_PER_W)
    gathered = _sc_gather(idx, table)
    return _mlp(gathered, W1, b1.reshape(1, HID), W2, b2.reshape(1, OUT))
